# baseline (speedup 1.0000x reference)
import functools

import jax
import jax.numpy as jnp
from jax import lax
from jax.experimental import pallas as pl
from jax.experimental.pallas import tpu as pltpu
from jax.experimental.pallas import tpu_sc as plsc

_B = 16384
_K = 64
_CH = 16
_NBUF = 4


@functools.lru_cache(maxsize=None)
def _make_lookup_kernel(B, K):
    info = plsc.get_sparse_core_info()
    nc, ns = info.num_cores, info.num_subcores
    nw = nc * ns
    bpw = B // nw
    n_ch = bpw // _CH
    assert bpw * nw == B and n_ch * _CH == bpw and n_ch % _NBUF == 0

    mesh = plsc.VectorSubcoreMesh(core_axis_name="c", subcore_axis_name="s")
    out_t = jax.ShapeDtypeStruct((B, K), jnp.float32)

    @functools.partial(
        pl.kernel,
        mesh=mesh,
        compiler_params=pltpu.CompilerParams(use_tc_tiling_on_sc=False),
        out_type=(out_t, out_t, out_t),
        scratch_types=[
            pltpu.VMEM((bpw,), jnp.int32),
            pltpu.VMEM((_NBUF, _CH, K), jnp.float32),
            pltpu.SemaphoreType.DMA,
            pltpu.SemaphoreType.DMA,
        ],
    )
    def lookup(idx_s, idx_p, idx_o, ent, rel, out_s, out_p, out_o,
               iv, sbuf, gsem, wsem):
        wid = lax.axis_index("s") * nc + lax.axis_index("c")
        base = wid * bpw

        def do_lookup(idx_hbm, tab, out_hbm):
            pltpu.sync_copy(idx_hbm.at[pl.ds(base, bpw)], iv)

            def fire_chunk(coff, b):
                v = iv[pl.ds(coff, _CH)]
                for r in range(_CH):
                    pltpu.async_copy(tab.at[v[r]], sbuf.at[b, r], gsem)

            for b in range(_NBUF):
                fire_chunk(b * _CH, b)

            def body(t, carry):
                for b in range(_NBUF):
                    c = t * _NBUF + b
                    coff = pl.multiple_of(c * _CH, _CH)
                    pltpu.make_async_copy(
                        out_hbm.at[pl.ds(0, _CH)], sbuf.at[b], gsem).wait()
                    pltpu.async_copy(
                        sbuf.at[b], out_hbm.at[pl.ds(base + coff, _CH)], wsem)
                    pltpu.make_async_copy(
                        sbuf.at[b], out_hbm.at[pl.ds(base, _CH)], wsem).wait()
                    @pl.when(c + _NBUF < n_ch)
                    def _():
                        noff = pl.multiple_of(
                            (t * _NBUF + b + _NBUF) * _CH, _CH)
                        fire_chunk(noff, b)
                return carry

            lax.fori_loop(0, n_ch // _NBUF, body, 0)

        do_lookup(idx_s, ent, out_s)
        do_lookup(idx_p, rel, out_p)
        do_lookup(idx_o, ent, out_o)

    return lookup


def kernel(x, ent_emb, rel_emb):
    xi = x.astype(jnp.int32)
    lookup = _make_lookup_kernel(_B, _K)
    return lookup(xi[:, 0], xi[:, 1], xi[:, 2], ent_emb, rel_emb)


# speedup vs baseline: 2.3283x; 2.3283x over previous
import functools

import jax
import jax.numpy as jnp
from jax import lax
from jax.experimental import pallas as pl
from jax.experimental.pallas import tpu as pltpu
from jax.experimental.pallas import tpu_sc as plsc

_B = 16384
_K = 64
_CH = 16
_NBUF = 4


@functools.lru_cache(maxsize=None)
def _make_lookup_kernel(B, K):
    info = plsc.get_sparse_core_info()
    nc, ns = info.num_cores, info.num_subcores
    nw = nc * ns
    bpw = B // nw
    n_ch = bpw // _CH
    assert bpw * nw == B and n_ch * _CH == bpw and n_ch % _NBUF == 0

    mesh = plsc.VectorSubcoreMesh(core_axis_name="c", subcore_axis_name="s")
    out_t = jax.ShapeDtypeStruct((B, K), jnp.float32)

    @functools.partial(
        pl.kernel,
        mesh=mesh,
        out_type=(out_t, out_t, out_t),
        scratch_types=[
            pltpu.VMEM((1, bpw), jnp.int32),
            pltpu.VMEM((_NBUF, _CH, K), jnp.float32),
            pltpu.SemaphoreType.DMA,
            pltpu.SemaphoreType.DMA,
        ],
    )
    def lookup(xt, ent, rel, out_s, out_p, out_o, iv, sbuf, gsem, wsem):
        wid = lax.axis_index("s") * nc + lax.axis_index("c")
        base = wid * bpw

        def do_lookup(col, tab, out_hbm):
            pltpu.sync_copy(xt.at[pl.ds(col, 1), pl.ds(base, bpw)], iv)

            def fire_chunk(coff, b):
                v = iv[0, pl.ds(coff, _CH)]
                blk = v >> 3
                sub = v & 7
                for r in range(_CH):
                    pltpu.async_copy(tab.at[blk[r], sub[r]], sbuf.at[b, r],
                                     gsem)

            for b in range(_NBUF):
                fire_chunk(b * _CH, b)

            def body(t, carry):
                for b in range(_NBUF):
                    c = t * _NBUF + b
                    coff = pl.multiple_of(c * _CH, _CH)
                    pltpu.make_async_copy(
                        out_hbm.at[pl.ds(0, _CH)], sbuf.at[b], gsem).wait()
                    pltpu.async_copy(
                        sbuf.at[b], out_hbm.at[pl.ds(base + coff, _CH)], wsem)
                    pltpu.make_async_copy(
                        sbuf.at[b], out_hbm.at[pl.ds(base, _CH)], wsem).wait()
                    @pl.when(c + _NBUF < n_ch)
                    def _():
                        noff = pl.multiple_of(
                            (t * _NBUF + b + _NBUF) * _CH, _CH)
                        fire_chunk(noff, b)
                return carry

            lax.fori_loop(0, n_ch // _NBUF, body, 0)

        do_lookup(0, ent, out_s)
        do_lookup(1, rel, out_p)
        do_lookup(2, ent, out_o)

    return lookup


def kernel(x, ent_emb, rel_emb):
    xt = x.astype(jnp.int32).T
    ent3 = ent_emb.reshape(ent_emb.shape[0] // 8, 8, _K)
    rel3 = rel_emb.reshape(rel_emb.shape[0] // 8, 8, _K)
    lookup = _make_lookup_kernel(_B, _K)
    return lookup(xt, ent3, rel3)


# speedup vs baseline: 2.3430x; 1.0063x over previous
import functools

import jax
import jax.numpy as jnp
from jax import lax
from jax.experimental import pallas as pl
from jax.experimental.pallas import tpu as pltpu
from jax.experimental.pallas import tpu_sc as plsc

_B = 16384
_K = 64
_CH = 16
_NBUF = 8


@functools.lru_cache(maxsize=None)
def _make_lookup_kernel(B, K):
    info = plsc.get_sparse_core_info()
    nc, ns = info.num_cores, info.num_subcores
    nw = nc * ns
    bpw = B // nw
    n_ch = bpw // _CH
    assert bpw * nw == B and n_ch * _CH == bpw and n_ch % _NBUF == 0

    mesh = plsc.VectorSubcoreMesh(core_axis_name="c", subcore_axis_name="s")
    out_t = jax.ShapeDtypeStruct((B, K), jnp.float32)

    @functools.partial(
        pl.kernel,
        mesh=mesh,
        out_type=(out_t, out_t, out_t),
        scratch_types=[
            pltpu.VMEM((1, bpw), jnp.int32),
            pltpu.VMEM((_NBUF, _CH, K), jnp.float32),
            pltpu.SemaphoreType.DMA,
            pltpu.SemaphoreType.DMA,
        ],
    )
    def lookup(xt, ent, rel, out_s, out_p, out_o, iv, sbuf, gsem, wsem):
        wid = lax.axis_index("s") * nc + lax.axis_index("c")
        base = wid * bpw

        def do_lookup(col, tab, out_hbm):
            pltpu.sync_copy(xt.at[pl.ds(col, 1), pl.ds(base, bpw)], iv)

            def fire_chunk(coff, b):
                v = iv[0, pl.ds(coff, _CH)]
                blk = v >> 3
                sub = v & 7
                for r in range(_CH):
                    pltpu.async_copy(tab.at[blk[r], sub[r]], sbuf.at[b, r],
                                     gsem)

            for b in range(_NBUF):
                fire_chunk(b * _CH, b)

            def body(t, carry):
                for b in range(_NBUF):
                    c = t * _NBUF + b
                    coff = pl.multiple_of(c * _CH, _CH)
                    pltpu.make_async_copy(
                        out_hbm.at[pl.ds(0, _CH)], sbuf.at[b], gsem).wait()
                    pltpu.async_copy(
                        sbuf.at[b], out_hbm.at[pl.ds(base + coff, _CH)], wsem)
                    pltpu.make_async_copy(
                        sbuf.at[b], out_hbm.at[pl.ds(base, _CH)], wsem).wait()
                    @pl.when(c + _NBUF < n_ch)
                    def _():
                        noff = pl.multiple_of(
                            (t * _NBUF + b + _NBUF) * _CH, _CH)
                        fire_chunk(noff, b)
                return carry

            lax.fori_loop(0, n_ch // _NBUF, body, 0)

        do_lookup(0, ent, out_s)
        do_lookup(1, rel, out_p)
        do_lookup(2, ent, out_o)

    return lookup


def kernel(x, ent_emb, rel_emb):
    xt = x.astype(jnp.int32).T
    ent3 = ent_emb.reshape(ent_emb.shape[0] // 8, 8, _K)
    rel3 = rel_emb.reshape(rel_emb.shape[0] // 8, 8, _K)
    lookup = _make_lookup_kernel(_B, _K)
    return lookup(xt, ent3, rel3)
